# baseline (speedup 1.0000x reference)
import functools

import jax
import jax.numpy as jnp
from jax import lax
from jax.experimental import pallas as pl
from jax.experimental.pallas import tpu as pltpu
from jax.experimental.pallas import tpu_sc as plsc

N = 10000
E = 320000
D = 128
L = 10
G = 64
C = 2

NC = 2
NS = 16
NW = NC * NS
CH = 128

GK = 8
J_E = GK * -(-E // (NW * CH * GK))
NG = J_E // GK
EP = NW * CH * J_E
J_N = -(-N // (NW * CH))
NPAD = NW * CH * J_N
NAGG = 10240
RPT = NAGG // NS

def _wid():
    return lax.axis_index("s") * NC + lax.axis_index("c")


def _sc_kernel(**kw):
    def deco(body):
        @functools.cache
        def build():
            mesh = plsc.VectorSubcoreMesh(core_axis_name="c",
                                          subcore_axis_name="s")
            return pl.kernel(body, mesh=mesh, **kw)
        return lambda *args: build()(*args)
    return deco


@_sc_kernel(
    out_type=jax.ShapeDtypeStruct((NPAD, D), jnp.float32),
    scratch_types=[
        pltpu.VMEM((J_N, CH), jnp.int32),
        pltpu.VMEM((CH, D), jnp.float32),
        pltpu.SemaphoreType.DMA,
    ],
)
def _emb_gather(xp_hbm, emb_hbm, out_hbm, idx_v, rows_v, sem):
    w = _wid()
    pltpu.sync_copy(xp_hbm.at[w], idx_v)
    for j in range(J_N):
        pltpu.async_copy(emb_hbm.at[idx_v.at[j]], rows_v, sem).wait()
        pltpu.sync_copy(rows_v, out_hbm.at[pl.ds((w * J_N + j) * CH, CH)])


@_sc_kernel(
    out_type=jax.ShapeDtypeStruct((NC, NAGG, D), jnp.float32),
    scratch_types=[
        pltpu.VMEM((J_E, CH), jnp.int32),
        pltpu.VMEM((J_E, CH), jnp.int32),
        pltpu.VMEM((CH, D), jnp.float32),
        pltpu.SemaphoreType.DMA,
        pltpu.VMEM_SHARED((NAGG, D), jnp.float32),
    ],
)
def _edge_agg(srcp_hbm, dstp_hbm, h_hbm, z_hbm, out_hbm,
              src_v, dst_v, rows_v, sem, agg_sh):
    c = lax.axis_index("c")
    s = lax.axis_index("s")
    w = s * NC + c
    pltpu.sync_copy(z_hbm.at[pl.ds(s * RPT, RPT)], agg_sh.at[pl.ds(s * RPT, RPT)])
    pltpu.sync_copy(srcp_hbm.at[w], src_v)
    pltpu.sync_copy(dstp_hbm.at[w], dst_v)
    plsc.subcore_barrier()

    def body(j, carry):
        pltpu.async_copy(h_hbm.at[src_v.at[j]], rows_v, sem).wait()
        pltpu.sync_copy(rows_v, agg_sh.at[dst_v.at[j]], add=True)
        return carry

    lax.fori_loop(0, J_E, body, 0)
    plsc.subcore_barrier()
    pltpu.sync_copy(agg_sh.at[pl.ds(s * RPT, RPT)],
                    out_hbm.at[c, pl.ds(s * RPT, RPT)])


EDC = 4096
HI = NAGG // 128


def _deg_body(dst_ref, out_ref):
    @pl.when(pl.program_id(0) == 0)
    def _():
        out_ref[...] = jnp.zeros((HI, 128), jnp.float32)

    d = dst_ref[...]
    hi = d // 128
    lo = d - hi * 128
    ohi = (lax.broadcasted_iota(jnp.int32, (HI, EDC), 0) == hi).astype(jnp.float32)
    oloT = (lax.broadcasted_iota(jnp.int32, (128, EDC), 0) == lo).astype(jnp.float32)
    out_ref[...] += lax.dot_general(ohi, oloT, (((1,), (1,)), ((), ())),
                                    preferred_element_type=jnp.float32)


_deg_call = pl.pallas_call(
    _deg_body,
    grid=(EP // EDC,),
    in_specs=[pl.BlockSpec((1, EDC), lambda i: (0, i))],
    out_specs=pl.BlockSpec((HI, 128), lambda i: (0, 0)),
    out_shape=jax.ShapeDtypeStruct((HI, 128), jnp.float32),
)


def _layer_body(part_ref, h_ref, deg_ref, wl_ref, wr_ref, blg_ref, out_ref):
    deg = deg_ref[...]
    agg = (part_ref[0, :N, :] + part_ref[1, :N, :]) / jnp.maximum(deg, 1.0)
    h = h_ref[:N, :]
    t = (jnp.dot(agg, wl_ref[...], preferred_element_type=jnp.float32)
         + jnp.dot(h, wr_ref[...], preferred_element_type=jnp.float32)
         + blg_ref[0:1, :])
    mean = jnp.mean(t, axis=0, keepdims=True)
    var = jnp.mean((t - mean) ** 2, axis=0, keepdims=True)
    hn = (t - mean) / jnp.sqrt(var + 1e-5) * blg_ref[1:2, :] + blg_ref[2:3, :]
    out_ref[:N, :] = jnp.maximum(hn, 0.0)
    out_ref[N:, :] = jnp.zeros((NPAD - N, D), jnp.float32)


def _pool_body(h_ref, mark_ref, wc_ref, bc_ref, out_ref):
    h = h_ref[:N, :]
    gid = lax.broadcasted_iota(jnp.int32, (G, N), 0)
    onehot = (gid == mark_ref[...]).astype(jnp.float32)
    pooled = jnp.dot(onehot, h, preferred_element_type=jnp.float32)
    counts = jnp.sum(onehot, axis=1, keepdims=True)
    pooled = pooled / jnp.maximum(counts, 1.0)
    out_ref[...] = jnp.dot(pooled, wc_ref[...],
                           preferred_element_type=jnp.float32) + bc_ref[...]


_layer_call = pl.pallas_call(
    _layer_body,
    out_shape=jax.ShapeDtypeStruct((NPAD, D), jnp.float32),
)

_pool_call = pl.pallas_call(
    _pool_body,
    out_shape=jax.ShapeDtypeStruct((G, C), jnp.float32),
)


def kernel(x, edge_index, u_index, example_mark, emb, Wl, bl, Wr, gamma,
           beta, Wc, bc):
    del u_index
    i32 = jnp.int32
    f32 = jnp.float32

    src = edge_index[0].astype(i32)
    dst = edge_index[1].astype(i32)
    srcp = jnp.concatenate([src, jnp.zeros((EP - E,), i32)]).reshape(NW, J_E, CH)
    dstp = jnp.concatenate([dst, jnp.full((EP - E,), N, i32)]).reshape(NW, J_E, CH)
    xp = jnp.concatenate([x.astype(i32), jnp.zeros((NPAD - N,), i32)]
                         ).reshape(NW, J_N, CH)
    zpad = jnp.zeros((NAGG, D), f32)
    blg = jnp.stack([bl, gamma, beta], axis=1)
    markp = example_mark.astype(i32).reshape(1, N)
    bc2 = bc.astype(f32).reshape(1, C)

    h = _emb_gather(xp, emb)
    deg = _deg_call(dstp.reshape(1, EP)).reshape(NAGG, 1)[:N]
    for i in range(L):
        part = _edge_agg(srcp, dstp, h, zpad)
        h = _layer_call(part, h, deg, Wl[i], Wr[i], blg[i])
    return _pool_call(h, markp, Wc, bc2)


# speedup vs baseline: 2.6823x; 2.6823x over previous
import functools

import jax
import jax.numpy as jnp
from jax import lax
from jax.experimental import pallas as pl
from jax.experimental.pallas import tpu as pltpu
from jax.experimental.pallas import tpu_sc as plsc

N = 10000
E = 320000
D = 128
L = 10
G = 64
C = 2

NC = 2
NS = 16
NW = NC * NS
CH = 128

GK = 8
J_E = GK * -(-E // (NW * CH * GK))
NG = J_E // GK
EP = NW * CH * J_E
J_N = -(-N // (NW * CH))
NPAD = NW * CH * J_N
NAGG = 10240
RPT = NAGG // NS

def _wid():
    return lax.axis_index("s") * NC + lax.axis_index("c")


def _sc_kernel(**kw):
    def deco(body):
        @functools.cache
        def build():
            mesh = plsc.VectorSubcoreMesh(core_axis_name="c",
                                          subcore_axis_name="s")
            return pl.kernel(body, mesh=mesh, **kw)
        return lambda *args: build()(*args)
    return deco


@_sc_kernel(
    out_type=jax.ShapeDtypeStruct((NPAD, D), jnp.float32),
    scratch_types=[
        pltpu.VMEM((J_N, CH), jnp.int32),
        pltpu.VMEM((CH, D), jnp.float32),
        pltpu.SemaphoreType.DMA,
    ],
)
def _emb_gather(xp_hbm, emb_hbm, out_hbm, idx_v, rows_v, sem):
    w = _wid()
    pltpu.sync_copy(xp_hbm.at[w], idx_v)
    for j in range(J_N):
        pltpu.async_copy(emb_hbm.at[idx_v.at[j]], rows_v, sem).wait()
        pltpu.sync_copy(rows_v, out_hbm.at[pl.ds((w * J_N + j) * CH, CH)])


@_sc_kernel(
    out_type=jax.ShapeDtypeStruct((NC, NAGG, D), jnp.float32),
    scratch_types=[
        pltpu.VMEM((J_E, CH), jnp.int32),
        pltpu.VMEM((J_E, CH), jnp.int32),
        pltpu.VMEM((CH, D), jnp.float32),
        pltpu.SemaphoreType.DMA,
        pltpu.VMEM_SHARED((NAGG, D), jnp.float32),
    ],
)
def _edge_agg(srcp_hbm, dstp_hbm, h_hbm, z_hbm, out_hbm,
              src_v, dst_v, rows_v, sem, agg_sh):
    c = lax.axis_index("c")
    s = lax.axis_index("s")
    w = s * NC + c
    pltpu.sync_copy(z_hbm.at[pl.ds(s * RPT, RPT)], agg_sh.at[pl.ds(s * RPT, RPT)])
    pltpu.sync_copy(srcp_hbm.at[w], src_v)
    pltpu.sync_copy(dstp_hbm.at[w], dst_v)
    plsc.subcore_barrier()

    def body(j, carry):
        pltpu.async_copy(h_hbm.at[src_v.at[j]], rows_v, sem).wait()
        pltpu.sync_copy(rows_v, agg_sh.at[dst_v.at[j]], add=True)
        return carry

    lax.fori_loop(0, J_E, body, 0)
    plsc.subcore_barrier()
    pltpu.sync_copy(agg_sh.at[pl.ds(s * RPT, RPT)],
                    out_hbm.at[c, pl.ds(s * RPT, RPT)])


EDC = 4096
HI = NAGG // 128


def _deg_body(dst_ref, out_ref):
    @pl.when(pl.program_id(0) == 0)
    def _():
        out_ref[...] = jnp.zeros((HI, 128), jnp.float32)

    d = dst_ref[...]
    hi = d // 128
    lo = d - hi * 128
    ohi = (lax.broadcasted_iota(jnp.int32, (HI, EDC), 0) == hi).astype(jnp.float32)
    oloT = (lax.broadcasted_iota(jnp.int32, (128, EDC), 0) == lo).astype(jnp.float32)
    out_ref[...] += lax.dot_general(ohi, oloT, (((1,), (1,)), ((), ())),
                                    preferred_element_type=jnp.float32)


_deg_call = pl.pallas_call(
    _deg_body,
    grid=(EP // EDC,),
    in_specs=[pl.BlockSpec((1, EDC), lambda i: (0, i))],
    out_specs=pl.BlockSpec((HI, 128), lambda i: (0, 0)),
    out_shape=jax.ShapeDtypeStruct((HI, 128), jnp.float32),
)


def _layer_body(part_ref, h_ref, deg_ref, wl_ref, wr_ref, blg_ref, out_ref):
    deg = deg_ref[...]
    agg = (part_ref[0, :N, :] + part_ref[1, :N, :]) / jnp.maximum(deg, 1.0)
    h = h_ref[:N, :]
    t = (jnp.dot(agg, wl_ref[...], preferred_element_type=jnp.float32)
         + jnp.dot(h, wr_ref[...], preferred_element_type=jnp.float32)
         + blg_ref[0:1, :])
    mean = jnp.mean(t, axis=0, keepdims=True)
    var = jnp.mean((t - mean) ** 2, axis=0, keepdims=True)
    hn = (t - mean) / jnp.sqrt(var + 1e-5) * blg_ref[1:2, :] + blg_ref[2:3, :]
    out_ref[:N, :] = jnp.maximum(hn, 0.0)
    out_ref[N:, :] = jnp.zeros((NPAD - N, D), jnp.float32)


def _pool_body(h_ref, mark_ref, wc_ref, bc_ref, out_ref):
    h = h_ref[:N, :]
    gid = lax.broadcasted_iota(jnp.int32, (G, N), 0)
    onehot = (gid == mark_ref[...]).astype(jnp.float32)
    pooled = jnp.dot(onehot, h, preferred_element_type=jnp.float32)
    counts = jnp.sum(onehot, axis=1, keepdims=True)
    pooled = pooled / jnp.maximum(counts, 1.0)
    out_ref[...] = jnp.dot(pooled, wc_ref[...],
                           preferred_element_type=jnp.float32) + bc_ref[...]


_layer_call = pl.pallas_call(
    _layer_body,
    out_shape=jax.ShapeDtypeStruct((NPAD, D), jnp.float32),
)

_pool_call = pl.pallas_call(
    _pool_body,
    out_shape=jax.ShapeDtypeStruct((G, C), jnp.float32),
)


def kernel(x, edge_index, u_index, example_mark, emb, Wl, bl, Wr, gamma,
           beta, Wc, bc):
    del u_index
    i32 = jnp.int32
    f32 = jnp.float32

    src = edge_index[0].astype(i32)
    dst = edge_index[1].astype(i32)
    pad_i = jnp.arange(EP - E, dtype=i32)
    srcp = jnp.concatenate([src, pad_i % N]).reshape(NW, J_E, CH)
    dstp = jnp.concatenate([dst, N + pad_i % (NAGG - N)]).reshape(NW, J_E, CH)
    xp = jnp.concatenate([x.astype(i32), jnp.zeros((NPAD - N,), i32)]
                         ).reshape(NW, J_N, CH)
    zpad = jnp.zeros((NAGG, D), f32)
    blg = jnp.stack([bl, gamma, beta], axis=1)
    markp = example_mark.astype(i32).reshape(1, N)
    bc2 = bc.astype(f32).reshape(1, C)

    h = _emb_gather(xp, emb)
    deg = _deg_call(dstp.reshape(1, EP)).reshape(NAGG, 1)[:N]
    for i in range(L):
        part = _edge_agg(srcp, dstp, h, zpad)
        h = _layer_call(part, h, deg, Wl[i], Wr[i], blg[i])
    return _pool_call(h, markp, Wc, bc2)


# speedup vs baseline: 3.4169x; 1.2738x over previous
import functools

import jax
import jax.numpy as jnp
from jax import lax
from jax.experimental import pallas as pl
from jax.experimental.pallas import tpu as pltpu
from jax.experimental.pallas import tpu_sc as plsc

N = 10000
E = 320000
D = 128
L = 10
G = 64
C = 2

NC = 2
NS = 16
NW = NC * NS
CH = 128

GK = 8
J_E = GK * -(-E // (NW * CH * GK))
NG = J_E // GK
EP = NW * CH * J_E
J_N = -(-N // (NW * CH))
NPAD = NW * CH * J_N
NAGG = 10240
RPT = NAGG // NS

def _wid():
    return lax.axis_index("s") * NC + lax.axis_index("c")


def _sc_kernel(**kw):
    def deco(body):
        @functools.cache
        def build():
            mesh = plsc.VectorSubcoreMesh(core_axis_name="c",
                                          subcore_axis_name="s")
            return pl.kernel(body, mesh=mesh, **kw)
        return lambda *args: build()(*args)
    return deco


@_sc_kernel(
    out_type=jax.ShapeDtypeStruct((NPAD, D), jnp.float32),
    scratch_types=[
        pltpu.VMEM((J_N, CH), jnp.int32),
        pltpu.VMEM((CH, D), jnp.float32),
        pltpu.SemaphoreType.DMA,
    ],
)
def _emb_gather(xp_hbm, emb_hbm, out_hbm, idx_v, rows_v, sem):
    w = _wid()
    pltpu.sync_copy(xp_hbm.at[w], idx_v)
    for j in range(J_N):
        pltpu.async_copy(emb_hbm.at[idx_v.at[j]], rows_v, sem).wait()
        pltpu.sync_copy(rows_v, out_hbm.at[pl.ds((w * J_N + j) * CH, CH)])


@_sc_kernel(
    out_type=jax.ShapeDtypeStruct((NC, NAGG, D), jnp.float32),
    scratch_types=[
        pltpu.VMEM((J_E // 2, CH), jnp.int32),
        pltpu.VMEM((J_E // 2, CH), jnp.int32),
        pltpu.VMEM((CH, D), jnp.float32),
        pltpu.VMEM((CH, D), jnp.float32),
        pltpu.SemaphoreType.DMA,
        pltpu.SemaphoreType.DMA,
        pltpu.SemaphoreType.DMA,
        pltpu.SemaphoreType.DMA,
        pltpu.VMEM_SHARED((NAGG, D), jnp.float32),
    ],
)
def _edge_agg(srcp_hbm, dstp_hbm, h_hbm, z_hbm, out_hbm,
              src_v, dst_v, rows0, rows1, g0, g1, s0, s1, agg_sh):
    c = lax.axis_index("c")
    s = lax.axis_index("s")
    w = s * NC + c
    JH = J_E // 2
    rows = (rows0, rows1)
    gs = (g0, g1)
    ss = (s0, s1)
    pltpu.sync_copy(z_hbm.at[pl.ds(s * RPT, RPT)], agg_sh.at[pl.ds(s * RPT, RPT)])
    pltpu.async_copy(h_hbm.at[pl.ds(0, CH)], rows0, s0)
    pltpu.async_copy(h_hbm.at[pl.ds(0, CH)], rows1, s1)
    plsc.subcore_barrier()

    def half(h2, carry):
        pltpu.sync_copy(srcp_hbm.at[w, pl.ds(h2 * JH, JH)], src_v)
        pltpu.sync_copy(dstp_hbm.at[w, pl.ds(h2 * JH, JH)], dst_v)

        def body(j2, carry2):
            for p in range(2):
                k = 2 * j2 + p
                pltpu.make_async_copy(h_hbm.at[pl.ds(0, CH)], rows[p], ss[p]).wait()
                pltpu.async_copy(h_hbm.at[src_v.at[k]], rows[p], gs[p]).wait()
                pltpu.async_copy(rows[p], agg_sh.at[dst_v.at[k]], ss[p], add=True)
            return carry2

        lax.fori_loop(0, JH // 2, body, 0)
        return carry

    lax.fori_loop(0, 2, half, 0)
    for p in range(2):
        pltpu.make_async_copy(h_hbm.at[pl.ds(0, CH)], rows[p], ss[p]).wait()
    plsc.subcore_barrier()
    pltpu.sync_copy(agg_sh.at[pl.ds(s * RPT, RPT)],
                    out_hbm.at[c, pl.ds(s * RPT, RPT)])


EDC = 4096
HI = NAGG // 128


def _deg_body(dst_ref, out_ref):
    @pl.when(pl.program_id(0) == 0)
    def _():
        out_ref[...] = jnp.zeros((HI, 128), jnp.float32)

    d = dst_ref[...]
    hi = d // 128
    lo = d - hi * 128
    ohi = (lax.broadcasted_iota(jnp.int32, (HI, EDC), 0) == hi).astype(jnp.float32)
    oloT = (lax.broadcasted_iota(jnp.int32, (128, EDC), 0) == lo).astype(jnp.float32)
    out_ref[...] += lax.dot_general(ohi, oloT, (((1,), (1,)), ((), ())),
                                    preferred_element_type=jnp.float32)


_deg_call = pl.pallas_call(
    _deg_body,
    grid=(EP // EDC,),
    in_specs=[pl.BlockSpec((1, EDC), lambda i: (0, i))],
    out_specs=pl.BlockSpec((HI, 128), lambda i: (0, 0)),
    out_shape=jax.ShapeDtypeStruct((HI, 128), jnp.float32),
)


def _layer_body(part_ref, h_ref, deg_ref, wl_ref, wr_ref, blg_ref, out_ref):
    deg = deg_ref[...]
    agg = (part_ref[0, :N, :] + part_ref[1, :N, :]) / jnp.maximum(deg, 1.0)
    h = h_ref[:N, :]
    t = (jnp.dot(agg, wl_ref[...], preferred_element_type=jnp.float32)
         + jnp.dot(h, wr_ref[...], preferred_element_type=jnp.float32)
         + blg_ref[0:1, :])
    mean = jnp.mean(t, axis=0, keepdims=True)
    var = jnp.mean((t - mean) ** 2, axis=0, keepdims=True)
    hn = (t - mean) / jnp.sqrt(var + 1e-5) * blg_ref[1:2, :] + blg_ref[2:3, :]
    out_ref[:N, :] = jnp.maximum(hn, 0.0)
    out_ref[N:, :] = jnp.zeros((NPAD - N, D), jnp.float32)


def _pool_body(h_ref, mark_ref, wc_ref, bc_ref, out_ref):
    h = h_ref[:N, :]
    gid = lax.broadcasted_iota(jnp.int32, (G, N), 0)
    onehot = (gid == mark_ref[...]).astype(jnp.float32)
    pooled = jnp.dot(onehot, h, preferred_element_type=jnp.float32)
    counts = jnp.sum(onehot, axis=1, keepdims=True)
    pooled = pooled / jnp.maximum(counts, 1.0)
    out_ref[...] = jnp.dot(pooled, wc_ref[...],
                           preferred_element_type=jnp.float32) + bc_ref[...]


_layer_call = pl.pallas_call(
    _layer_body,
    out_shape=jax.ShapeDtypeStruct((NPAD, D), jnp.float32),
)

_pool_call = pl.pallas_call(
    _pool_body,
    out_shape=jax.ShapeDtypeStruct((G, C), jnp.float32),
)


def kernel(x, edge_index, u_index, example_mark, emb, Wl, bl, Wr, gamma,
           beta, Wc, bc):
    del u_index
    i32 = jnp.int32
    f32 = jnp.float32

    src = edge_index[0].astype(i32)
    dst = edge_index[1].astype(i32)
    pad_i = jnp.arange(EP - E, dtype=i32)
    srcp = jnp.concatenate([src, pad_i % N]).reshape(NW, J_E, CH)
    dstp = jnp.concatenate([dst, N + pad_i % (NAGG - N)]).reshape(NW, J_E, CH)
    xp = jnp.concatenate([x.astype(i32), jnp.zeros((NPAD - N,), i32)]
                         ).reshape(NW, J_N, CH)
    zpad = jnp.zeros((NAGG, D), f32)
    blg = jnp.stack([bl, gamma, beta], axis=1)
    markp = example_mark.astype(i32).reshape(1, N)
    bc2 = bc.astype(f32).reshape(1, C)

    h = _emb_gather(xp, emb)
    deg = _deg_call(dstp.reshape(1, EP)).reshape(NAGG, 1)[:N]
    for i in range(L):
        part = _edge_agg(srcp, dstp, h, zpad)
        h = _layer_call(part, h, deg, Wl[i], Wr[i], blg[i])
    return _pool_call(h, markp, Wc, bc2)
